# baseline (speedup 1.0000x reference)
import jax
import jax.numpy as jnp
from jax.experimental import pallas as pl
from jax.experimental.pallas import tpu as pltpu

_TOPK = 8
_S, _B, _C, _D = 2048, 4, 64, 1024
_SC = 512
_NCHUNK = _S // _SC


def _eighth_largest(s):
    neg_inf = jnp.float32(-jnp.inf)
    shp = (s.shape[0], 1)
    thr = jnp.full(shp, jnp.inf, jnp.float32)
    ans = jnp.full(shp, -jnp.inf, jnp.float32)
    k = jnp.full(shp, _TOPK, jnp.int32)
    done = jnp.zeros(shp, jnp.bool_)
    for _ in range(_TOPK):
        cand = jnp.where(s < thr, s, neg_inf)
        m = jnp.max(cand, axis=1, keepdims=True)
        c = jnp.sum((s == m).astype(jnp.int32), axis=1, keepdims=True)
        newly = jnp.logical_and(jnp.logical_not(done), k <= c)
        ans = jnp.where(newly, m, ans)
        cont = jnp.logical_not(jnp.logical_or(done, newly))
        k = jnp.where(cont, k - c, k)
        thr = jnp.where(cont, m, thr)
        done = jnp.logical_or(done, newly)
    return ans


def _fused(sim_ref, x_ref, pad_ref, pl_ref, out_ref, m_ref, den_ref):
    j = pl.program_id(0)
    neg_inf = jnp.float32(-jnp.inf)

    @pl.when(j == 0)
    def _init():
        m_ref[...] = jnp.zeros((_B, _C), jnp.float32)
        den_ref[...] = jnp.zeros((_B, _C), jnp.float32)

    pad = pad_ref[...]
    for b in range(_B):
        s = sim_ref[:, b, :]
        s = jnp.where(pad[:, b:b + 1] > 0, neg_inf, s)
        s = jnp.where(s > 0, s, neg_inf)
        t = _eighth_largest(s)
        masked = jnp.where(s >= t, s, neg_inf)
        pl_ref[:, b, :] = (masked > 0).astype(jnp.float32)
        m_chunk = jnp.maximum(jnp.max(masked, axis=0, keepdims=True), 0.0)
        m_old = m_ref[b:b + 1, :]
        m_new = jnp.maximum(m_old, m_chunk)
        scale = jnp.exp(m_old - m_new)
        p = jnp.exp(masked - m_new)
        den_ref[b:b + 1, :] = (den_ref[b:b + 1, :] * scale
                               + jnp.sum(p, axis=0, keepdims=True))
        m_ref[b:b + 1, :] = m_new
        part = jax.lax.dot_general(
            p, x_ref[:, b, :],
            dimension_numbers=(((0,), (0,)), ((), ())),
            precision=jax.lax.Precision.HIGHEST,
            preferred_element_type=jnp.float32)
        prev = jnp.where(j == 0, jnp.zeros((_C, _D), jnp.float32),
                         out_ref[:, b, :])
        out_ref[:, b, :] = prev * jnp.transpose(scale) + part

    @pl.when(j == _NCHUNK - 1)
    def _fin():
        den = den_ref[...]
        inv = jnp.where(den > 0, 1.0 / den, 0.0)
        for b in range(_B):
            out_ref[:, b, :] = (out_ref[:, b, :]
                                * jnp.transpose(inv[b:b + 1, :]))


def kernel(x, token_concept_embedding, key_padding_mask):
    padT = key_padding_mask.astype(jnp.float32).T
    pseudo_label, merge_val = pl.pallas_call(
        _fused,
        grid=(_NCHUNK,),
        in_specs=[
            pl.BlockSpec((_SC, _B, _C), lambda j: (j, 0, 0)),
            pl.BlockSpec((_SC, _B, _D), lambda j: (j, 0, 0)),
            pl.BlockSpec((_SC, _B), lambda j: (j, 0)),
        ],
        out_specs=[
            pl.BlockSpec((_SC, _B, _C), lambda j: (j, 0, 0)),
            pl.BlockSpec((_C, _B, _D), lambda j: (0, 0, 0)),
        ],
        out_shape=[
            jax.ShapeDtypeStruct((_S, _B, _C), jnp.float32),
            jax.ShapeDtypeStruct((_C, _B, _D), jnp.float32),
        ],
        scratch_shapes=[
            pltpu.VMEM((_B, _C), jnp.float32),
            pltpu.VMEM((_B, _C), jnp.float32),
        ],
        compiler_params=pltpu.CompilerParams(
            dimension_semantics=("arbitrary",),
        ),
    )(token_concept_embedding, x, padT)
    return merge_val, pseudo_label


# speedup vs baseline: 2.0389x; 2.0389x over previous
import jax
import jax.numpy as jnp
from jax.experimental import pallas as pl
from jax.experimental.pallas import tpu as pltpu

_TOPK = 8
_S, _B, _C, _D = 2048, 4, 64, 1024
_SC = 512
_NCHUNK = _S // _SC


def _eighth_largest(s):
    neg_inf = jnp.float32(-jnp.inf)
    shp = (1, s.shape[1])
    thr = jnp.full(shp, jnp.inf, jnp.float32)
    ans = jnp.full(shp, -jnp.inf, jnp.float32)
    k = jnp.full(shp, float(_TOPK), jnp.float32)
    done = jnp.zeros(shp, jnp.bool_)
    for _ in range(_TOPK):
        cand = jnp.where(s < thr, s, neg_inf)
        m = jnp.max(cand, axis=0, keepdims=True)
        c = jnp.sum(jnp.where(s == m, 1.0, 0.0), axis=0, keepdims=True)
        newly = jnp.logical_and(jnp.logical_not(done), k <= c)
        ans = jnp.where(newly, m, ans)
        cont = jnp.logical_not(jnp.logical_or(done, newly))
        k = jnp.where(cont, k - c, k)
        thr = jnp.where(cont, m, thr)
        done = jnp.logical_or(done, newly)
    return ans


def _fused(sim_ref, x_ref, pad_ref, pl_ref, out_ref, m_ref, den_ref):
    j = pl.program_id(0)
    neg_inf = jnp.float32(-jnp.inf)

    @pl.when(j == 0)
    def _init():
        m_ref[...] = jnp.zeros((_B, _C), jnp.float32)
        den_ref[...] = jnp.zeros((_B, _C), jnp.float32)

    for b in range(_B):
        s = jnp.transpose(sim_ref[:, b, :])
        pad = pad_ref[b:b + 1, :]
        s = jnp.where(pad > 0, neg_inf, s)
        s = jnp.where(s > 0, s, neg_inf)
        t = _eighth_largest(s)
        masked = jnp.where(s >= t, s, neg_inf)
        pl_ref[:, b, :] = jnp.transpose((masked > 0).astype(jnp.float32))
        m_chunk = jnp.maximum(jnp.max(masked, axis=1, keepdims=True), 0.0)
        m_old = jnp.transpose(m_ref[b:b + 1, :])
        m_new = jnp.maximum(m_old, m_chunk)
        scale = jnp.exp(m_old - m_new)
        p = jnp.exp(masked - m_new)
        den_old = jnp.transpose(den_ref[b:b + 1, :])
        den_new = den_old * scale + jnp.sum(p, axis=1, keepdims=True)
        den_ref[b:b + 1, :] = jnp.transpose(den_new)
        m_ref[b:b + 1, :] = jnp.transpose(m_new)
        part = jax.lax.dot(
            p, x_ref[:, b, :],
            precision=jax.lax.Precision.DEFAULT,
            preferred_element_type=jnp.float32)
        prev = jnp.where(j == 0, jnp.zeros((_C, _D), jnp.float32),
                         out_ref[:, b, :])
        out_ref[:, b, :] = prev * scale + part

    @pl.when(j == _NCHUNK - 1)
    def _fin():
        den = den_ref[...]
        inv = jnp.where(den > 0, 1.0 / den, 0.0)
        for b in range(_B):
            out_ref[:, b, :] = (out_ref[:, b, :]
                                * jnp.transpose(inv[b:b + 1, :]))


def kernel(x, token_concept_embedding, key_padding_mask):
    padf = key_padding_mask.astype(jnp.float32)
    pseudo_label, merge_val = pl.pallas_call(
        _fused,
        grid=(_NCHUNK,),
        in_specs=[
            pl.BlockSpec((_SC, _B, _C), lambda j: (j, 0, 0)),
            pl.BlockSpec((_SC, _B, _D), lambda j: (j, 0, 0)),
            pl.BlockSpec((_B, _SC), lambda j: (0, j)),
        ],
        out_specs=[
            pl.BlockSpec((_SC, _B, _C), lambda j: (j, 0, 0)),
            pl.BlockSpec((_C, _B, _D), lambda j: (0, 0, 0)),
        ],
        out_shape=[
            jax.ShapeDtypeStruct((_S, _B, _C), jnp.float32),
            jax.ShapeDtypeStruct((_C, _B, _D), jnp.float32),
        ],
        scratch_shapes=[
            pltpu.VMEM((_B, _C), jnp.float32),
            pltpu.VMEM((_B, _C), jnp.float32),
        ],
        compiler_params=pltpu.CompilerParams(
            dimension_semantics=("arbitrary",),
        ),
    )(token_concept_embedding, x, padf)
    return merge_val, pseudo_label
